# Initial kernel scaffold; baseline (speedup 1.0000x reference)
import jax
import jax.numpy as jnp
from jax.experimental import pallas as pl


def kernel(x, gam0, gam1, gam2, edge_index, batch, Wl0, Wr0, bc0, bng0, bnb0, Wl1, Wr1, bc1, bng1, bnb1, Wl2, Wr2, bc2, bng2, bnb2, Wl3, Wr3, bc3, bng3, bnb3, fc_W, fc_b, mlp_W1, mlp_b1, mlp_W2, mlp_b2):
    raise NotImplementedError("write your pallas kernel here")


# speedup vs baseline: 1.0100x; 1.0100x over previous
import functools

import jax
import jax.numpy as jnp
from jax import lax
from jax.experimental import pallas as pl
from jax.experimental.pallas import tpu as pltpu

N = 10000
E = 320000
H = 512
G = 16
OUT = 128
NB = 10
R = N // NB


def _mm_stats_body(h_ref, agg_ref, wl_ref, wr_ref, bc_ref, u_ref, stats_ref):
    i = pl.program_id(0)
    h = h_ref[...]
    a = agg_ref[...]
    u = lax.dot_general(h, wl_ref[...], (((1,), (1,)), ((), ())),
                        preferred_element_type=jnp.float32)
    u += lax.dot_general(a, wr_ref[...], (((1,), (1,)), ((), ())),
                         preferred_element_type=jnp.float32)
    u += bc_ref[...]
    u_ref[...] = u
    s1 = jnp.sum(u, axis=0, keepdims=True)
    s2 = jnp.sum(u * u, axis=0, keepdims=True)
    new = jnp.concatenate([s1, s2, jnp.zeros((6, H), jnp.float32)], axis=0)

    @pl.when(i == 0)
    def _():
        stats_ref[...] = new

    @pl.when(i > 0)
    def _():
        stats_ref[...] += new


def _mm_stats(h, agg_scaled, wl, wr, bc):
    return pl.pallas_call(
        _mm_stats_body,
        grid=(NB,),
        in_specs=[
            pl.BlockSpec((R, H), lambda i: (i, 0)),
            pl.BlockSpec((R, H), lambda i: (i, 0)),
            pl.BlockSpec((H, H), lambda i: (0, 0)),
            pl.BlockSpec((H, H), lambda i: (0, 0)),
            pl.BlockSpec((1, H), lambda i: (0, 0)),
        ],
        out_specs=[
            pl.BlockSpec((R, H), lambda i: (i, 0)),
            pl.BlockSpec((8, H), lambda i: (0, 0)),
        ],
        out_shape=[
            jax.ShapeDtypeStruct((N, H), jnp.float32),
            jax.ShapeDtypeStruct((8, H), jnp.float32),
        ],
    )(h, agg_scaled, wl, wr, bc)


def _bn_body(u_ref, stats_ref, g_ref, b_ref, o_ref):
    u = u_ref[...]
    mu = stats_ref[0:1, :] * (1.0 / N)
    var = stats_ref[1:2, :] * (1.0 / N) - mu * mu
    inv = lax.rsqrt(var + 1e-5)
    o_ref[...] = jnp.maximum((u - mu) * inv * g_ref[...] + b_ref[...], 0.0)


def _bn_relu(u, stats, g, b):
    return pl.pallas_call(
        _bn_body,
        grid=(NB,),
        in_specs=[
            pl.BlockSpec((R, H), lambda i: (i, 0)),
            pl.BlockSpec((8, H), lambda i: (0, 0)),
            pl.BlockSpec((1, H), lambda i: (0, 0)),
            pl.BlockSpec((1, H), lambda i: (0, 0)),
        ],
        out_specs=pl.BlockSpec((R, H), lambda i: (i, 0)),
        out_shape=jax.ShapeDtypeStruct((N, H), jnp.float32),
    )(u, stats, g, b)


def _head_body(h_ref, batch_ref, fcw_ref, fcb_ref, w1_ref, b1_ref,
               w2_ref, b2_ref, o_ref, pooled_acc, cnt_acc):
    i = pl.program_id(0)
    b = batch_ref[0, 0, :]
    onehot = (b[:, None] == lax.broadcasted_iota(jnp.int32, (1, G), 1)
              ).astype(jnp.float32)
    pooled = lax.dot_general(onehot, h_ref[...], (((0,), (0,)), ((), ())),
                             preferred_element_type=jnp.float32)
    cnt = jnp.sum(onehot, axis=0, keepdims=True)

    @pl.when(i == 0)
    def _():
        pooled_acc[...] = pooled
        cnt_acc[...] = cnt

    @pl.when(i > 0)
    def _():
        pooled_acc[...] += pooled
        cnt_acc[...] += cnt

    @pl.when(i == NB - 1)
    def _():
        p = pooled_acc[...] / jnp.maximum(cnt_acc[...], 1.0).reshape(G, 1)
        t = lax.dot_general(p, fcw_ref[...], (((1,), (1,)), ((), ())),
                            preferred_element_type=jnp.float32)
        t = jnp.maximum(t + fcb_ref[...], 0.0)
        t = lax.dot_general(t, w1_ref[...], (((1,), (1,)), ((), ())),
                            preferred_element_type=jnp.float32)
        t = jnp.maximum(t + b1_ref[...], 0.0)
        t = lax.dot_general(t, w2_ref[...], (((1,), (1,)), ((), ())),
                            preferred_element_type=jnp.float32)
        o_ref[...] = t + b2_ref[...]


def _pool_head(h, batch3d, fc_W, fc_b, w1, b1, w2, b2):
    return pl.pallas_call(
        _head_body,
        grid=(NB,),
        in_specs=[
            pl.BlockSpec((R, H), lambda i: (i, 0)),
            pl.BlockSpec((1, 1, R), lambda i: (i, 0, 0)),
            pl.BlockSpec((H, H), lambda i: (0, 0)),
            pl.BlockSpec((1, H), lambda i: (0, 0)),
            pl.BlockSpec((H, H), lambda i: (0, 0)),
            pl.BlockSpec((1, H), lambda i: (0, 0)),
            pl.BlockSpec((OUT, H), lambda i: (0, 0)),
            pl.BlockSpec((1, OUT), lambda i: (0, 0)),
        ],
        out_specs=pl.BlockSpec((G, OUT), lambda i: (0, 0)),
        out_shape=jax.ShapeDtypeStruct((G, OUT), jnp.float32),
        scratch_shapes=[
            pltpu.VMEM((G, H), jnp.float32),
            pltpu.VMEM((1, G), jnp.float32),
        ],
    )(h, batch3d, fc_W, fc_b, w1, b1, w2, b2)


def kernel(x, gam0, gam1, gam2, edge_index, batch,
           Wl0, Wr0, bc0, bng0, bnb0,
           Wl1, Wr1, bc1, bng1, bnb1,
           Wl2, Wr2, bc2, bng2, bnb2,
           Wl3, Wr3, bc3, bng3, bnb3,
           fc_W, fc_b, mlp_W1, mlp_b1, mlp_W2, mlp_b2):
    h = jnp.concatenate([x, gam0, gam1, gam2], axis=1)
    src = edge_index[0].astype(jnp.int32)
    dst = edge_index[1].astype(jnp.int32)
    deg = jax.ops.segment_sum(jnp.ones((E,), jnp.float32), dst, num_segments=N)
    deginv = 1.0 / jnp.clip(deg, 1.0)
    batch3d = batch.astype(jnp.int32).reshape(NB, 1, R)

    convs = [(Wl0, Wr0, bc0, bng0, bnb0), (Wl1, Wr1, bc1, bng1, bnb1),
             (Wl2, Wr2, bc2, bng2, bnb2), (Wl3, Wr3, bc3, bng3, bnb3)]
    for (Wl, Wr, bc, g, b) in convs:
        agg = jax.ops.segment_sum(h[src], dst, num_segments=N)
        agg_scaled = agg * deginv[:, None]
        u, stats = _mm_stats(h, agg_scaled, Wl, Wr,
                             bc.reshape(1, H))
        h = _bn_relu(u, stats, g.reshape(1, H), b.reshape(1, H))

    return _pool_head(h, batch3d, fc_W, fc_b.reshape(1, H),
                      mlp_W1, mlp_b1.reshape(1, H),
                      mlp_W2, mlp_b2.reshape(1, OUT))


# speedup vs baseline: 1.4380x; 1.4238x over previous
import dataclasses
import functools

import jax
import jax.numpy as jnp
from jax import lax
from jax.experimental import pallas as pl
from jax.experimental.pallas import tpu as pltpu
from jax.experimental.pallas import tpu_sc as plsc

N = 10000
E = 320000
H = 512
G = 16
OUT = 128
NB = 10
R = N // NB

NC = 2
NS = 16
NW = NC * NS
ES = E // NS
RNG = 160
NR = 2
NRANGES = NW * NR
AGGR = NRANGES * RNG
BATCH = 64
SELCAP = 20480
FCH = 1024

_vector_mesh = plsc.VectorSubcoreMesh(core_axis_name="c", subcore_axis_name="s")

_sc_params = pltpu.CompilerParams()
if "needs_layout_passes" in pltpu.CompilerParams.__dataclass_fields__:
    _sc_params = dataclasses.replace(_sc_params, needs_layout_passes=False)


def _bucket_body(src_hbm, dst_hbm, bsrc_hbm, bdst_hbm, cnts_hbm, degp_hbm,
                 sbuf, dbuf, sel_s, sel_d, deg_l, cntv):
    cid = lax.axis_index("c")
    sid = lax.axis_index("s")
    w = cid * NS + sid

    zero16f = jnp.zeros((16,), jnp.float32)
    ones16 = jnp.ones((16,), jnp.float32)
    zero16i = jnp.zeros((16,), jnp.int32)
    sent16 = jnp.full((16,), RNG, jnp.int32)
    iota16 = lax.iota(jnp.int32, 16)

    for r in range(NR):
        rid = r * NW + w
        lo = rid * RNG

        @pl.loop(0, RNG // 16)
        def _(i):
            deg_l[pl.ds(i * 16, 16)] = zero16f

        def span_step(s, cv, lo=lo, rid=rid):
            pltpu.sync_copy(src_hbm.at[pl.ds(s * ES, ES)], sbuf)
            pltpu.sync_copy(dst_hbm.at[pl.ds(s * ES, ES)], dbuf)

            def step(i, cur):
                sv = sbuf[pl.ds(i * 16, 16)]
                dv = dbuf[pl.ds(i * 16, 16)] - lo
                m = (dv >= 0) & (dv < RNG)
                plsc.store_compressed(sel_s.at[pl.ds(cur, 16)], sv, mask=m)
                plsc.store_compressed(sel_d.at[pl.ds(cur, 16)], dv, mask=m)
                plsc.addupdate_scatter(deg_l, [dv], ones16, mask=m)
                return cur + jnp.max(plsc.all_reduce_population_count(m))

            cur = lax.fori_loop(0, ES // 16, step, jnp.int32(0))
            for k in range(BATCH // 16):
                sel_s[pl.ds(cur + k * 16, 16)] = zero16i
                sel_d[pl.ds(cur + k * 16, 16)] = sent16
            npad = ((cur + BATCH - 1) // BATCH) * BATCH

            def flush(j, carry):
                pltpu.sync_copy(sel_s.at[pl.ds(j * FCH, FCH)],
                                bsrc_hbm.at[rid, s, pl.ds(j * FCH, FCH)])
                pltpu.sync_copy(sel_d.at[pl.ds(j * FCH, FCH)],
                                bdst_hbm.at[rid, s, pl.ds(j * FCH, FCH)])
                return carry

            lax.fori_loop(0, (npad + FCH - 1) // FCH, flush, jnp.int32(0))
            return jnp.where(iota16 == s, npad, cv)

        cv = lax.fori_loop(0, NS, span_step, zero16i)
        cntv[...] = cv
        pltpu.sync_copy(cntv, cnts_hbm.at[rid])
        pltpu.sync_copy(deg_l, degp_hbm.at[rid])


@functools.partial(
    pl.kernel,
    out_type=(
        jax.ShapeDtypeStruct((NRANGES, NS, SELCAP), jnp.int32),
        jax.ShapeDtypeStruct((NRANGES, NS, SELCAP), jnp.int32),
        jax.ShapeDtypeStruct((NRANGES, 16), jnp.int32),
        jax.ShapeDtypeStruct((NRANGES, RNG), jnp.float32),
    ),
    mesh=_vector_mesh,
    scratch_types=[
        pltpu.VMEM((ES,), jnp.int32),
        pltpu.VMEM((ES,), jnp.int32),
        pltpu.VMEM((SELCAP,), jnp.int32),
        pltpu.VMEM((SELCAP,), jnp.int32),
        pltpu.VMEM((RNG,), jnp.float32),
        pltpu.VMEM((16,), jnp.int32),
    ],
    compiler_params=_sc_params,
)
def _bucket(*args):
    _bucket_body(*args)


def _agg_body(h_hbm, bsrc_hbm, bdst_hbm, cnts_hbm, agg_hbm,
              acc, rows, sidx, didx, cntv, sem):
    cid = lax.axis_index("c")
    sid = lax.axis_index("s")
    w = cid * NS + sid
    iota16 = lax.iota(jnp.int32, 16)
    zero16f = jnp.zeros((16,), jnp.float32)

    for r in range(NR):
        rid = r * NW + w
        lo = rid * RNG

        @pl.loop(0, (RNG + 1) * H // 16)
        def _(i):
            acc[pl.ds(i * 16, 16)] = zero16f

        pltpu.sync_copy(cnts_hbm.at[rid], cntv)
        cvec = cntv[...]

        def span_step(s, carry, rid=rid):
            n = jnp.max(jnp.where(iota16 == s, cvec, 0))

            def batch_step(j, c2):
                off = j * BATCH
                pltpu.sync_copy(bsrc_hbm.at[rid, s, pl.ds(off, BATCH)], sidx)
                pltpu.sync_copy(bdst_hbm.at[rid, s, pl.ds(off, BATCH)],
                                didx.at[pl.ds(0, BATCH)])
                pltpu.async_copy(h_hbm.at[sidx], rows, sem).wait()

                def edge_step(e, c3):
                    base = didx[pl.ds(e, 16)][0] * H
                    for k in range(H // 16):
                        plsc.addupdate(acc.at[pl.ds(base + k * 16, 16)],
                                       rows[e, pl.ds(k * 16, 16)])
                    return c3

                lax.fori_loop(0, BATCH, edge_step, jnp.int32(0))
                return c2

            lax.fori_loop(0, n // BATCH, batch_step, jnp.int32(0))
            return carry

        lax.fori_loop(0, NS, span_step, jnp.int32(0))
        pltpu.sync_copy(acc.at[pl.ds(0, RNG * H)],
                        agg_hbm.at[pl.ds(lo * H, RNG * H)])


@functools.partial(
    pl.kernel,
    out_type=jax.ShapeDtypeStruct((AGGR * H,), jnp.float32),
    mesh=_vector_mesh,
    scratch_types=[
        pltpu.VMEM(((RNG + 1) * H,), jnp.float32),
        pltpu.VMEM((BATCH, H), jnp.float32),
        pltpu.VMEM((BATCH,), jnp.int32),
        pltpu.VMEM((BATCH + 16,), jnp.int32),
        pltpu.VMEM((16,), jnp.int32),
        pltpu.SemaphoreType.DMA,
    ],
    compiler_params=_sc_params,
)
def _agg(*args):
    _agg_body(*args)


def _mm_stats_body(h_ref, agg_ref, dg_ref, wl_ref, wr_ref, bc_ref,
                   u_ref, stats_ref):
    i = pl.program_id(0)
    h = h_ref[...]
    dg = dg_ref[0, 0, :]
    a = agg_ref[...] * dg[:, None]
    u = lax.dot_general(h, wl_ref[...], (((1,), (1,)), ((), ())),
                        preferred_element_type=jnp.float32)
    u += lax.dot_general(a, wr_ref[...], (((1,), (1,)), ((), ())),
                         preferred_element_type=jnp.float32)
    u += bc_ref[...]
    u_ref[...] = u
    s1 = jnp.sum(u, axis=0, keepdims=True)
    s2 = jnp.sum(u * u, axis=0, keepdims=True)
    new = jnp.concatenate([s1, s2, jnp.zeros((6, H), jnp.float32)], axis=0)

    @pl.when(i == 0)
    def _():
        stats_ref[...] = new

    @pl.when(i > 0)
    def _():
        stats_ref[...] += new


def _mm_stats(h, agg, deginv3d, wl, wr, bc):
    return pl.pallas_call(
        _mm_stats_body,
        grid=(NB,),
        in_specs=[
            pl.BlockSpec((R, H), lambda i: (i, 0)),
            pl.BlockSpec((R, H), lambda i: (i, 0)),
            pl.BlockSpec((1, 1, R), lambda i: (i, 0, 0)),
            pl.BlockSpec((H, H), lambda i: (0, 0)),
            pl.BlockSpec((H, H), lambda i: (0, 0)),
            pl.BlockSpec((1, H), lambda i: (0, 0)),
        ],
        out_specs=[
            pl.BlockSpec((R, H), lambda i: (i, 0)),
            pl.BlockSpec((8, H), lambda i: (0, 0)),
        ],
        out_shape=[
            jax.ShapeDtypeStruct((N, H), jnp.float32),
            jax.ShapeDtypeStruct((8, H), jnp.float32),
        ],
    )(h, agg, deginv3d, wl, wr, bc)


def _bn_body(u_ref, stats_ref, g_ref, b_ref, o_ref):
    u = u_ref[...]
    mu = stats_ref[0:1, :] * (1.0 / N)
    var = stats_ref[1:2, :] * (1.0 / N) - mu * mu
    inv = lax.rsqrt(var + 1e-5)
    o_ref[...] = jnp.maximum((u - mu) * inv * g_ref[...] + b_ref[...], 0.0)


def _bn_relu(u, stats, g, b):
    return pl.pallas_call(
        _bn_body,
        grid=(NB,),
        in_specs=[
            pl.BlockSpec((R, H), lambda i: (i, 0)),
            pl.BlockSpec((8, H), lambda i: (0, 0)),
            pl.BlockSpec((1, H), lambda i: (0, 0)),
            pl.BlockSpec((1, H), lambda i: (0, 0)),
        ],
        out_specs=pl.BlockSpec((R, H), lambda i: (i, 0)),
        out_shape=jax.ShapeDtypeStruct((N, H), jnp.float32),
    )(u, stats, g, b)


def _head_body(h_ref, batch_ref, fcw_ref, fcb_ref, w1_ref, b1_ref,
               w2_ref, b2_ref, o_ref, pooled_acc, cnt_acc):
    i = pl.program_id(0)
    b = batch_ref[0, 0, :]
    onehot = (b[:, None] == lax.broadcasted_iota(jnp.int32, (1, G), 1)
              ).astype(jnp.float32)
    pooled = lax.dot_general(onehot, h_ref[...], (((0,), (0,)), ((), ())),
                             preferred_element_type=jnp.float32)
    cnt = jnp.sum(onehot, axis=0, keepdims=True)

    @pl.when(i == 0)
    def _():
        pooled_acc[...] = pooled
        cnt_acc[...] = cnt

    @pl.when(i > 0)
    def _():
        pooled_acc[...] += pooled
        cnt_acc[...] += cnt

    @pl.when(i == NB - 1)
    def _():
        p = pooled_acc[...] / jnp.maximum(cnt_acc[...], 1.0).reshape(G, 1)
        t = lax.dot_general(p, fcw_ref[...], (((1,), (1,)), ((), ())),
                            preferred_element_type=jnp.float32)
        t = jnp.maximum(t + fcb_ref[...], 0.0)
        t = lax.dot_general(t, w1_ref[...], (((1,), (1,)), ((), ())),
                            preferred_element_type=jnp.float32)
        t = jnp.maximum(t + b1_ref[...], 0.0)
        t = lax.dot_general(t, w2_ref[...], (((1,), (1,)), ((), ())),
                            preferred_element_type=jnp.float32)
        o_ref[...] = t + b2_ref[...]


def _pool_head(h, batch3d, fc_W, fc_b, w1, b1, w2, b2):
    return pl.pallas_call(
        _head_body,
        grid=(NB,),
        in_specs=[
            pl.BlockSpec((R, H), lambda i: (i, 0)),
            pl.BlockSpec((1, 1, R), lambda i: (i, 0, 0)),
            pl.BlockSpec((H, H), lambda i: (0, 0)),
            pl.BlockSpec((1, H), lambda i: (0, 0)),
            pl.BlockSpec((H, H), lambda i: (0, 0)),
            pl.BlockSpec((1, H), lambda i: (0, 0)),
            pl.BlockSpec((OUT, H), lambda i: (0, 0)),
            pl.BlockSpec((1, OUT), lambda i: (0, 0)),
        ],
        out_specs=pl.BlockSpec((G, OUT), lambda i: (0, 0)),
        out_shape=jax.ShapeDtypeStruct((G, OUT), jnp.float32),
        scratch_shapes=[
            pltpu.VMEM((G, H), jnp.float32),
            pltpu.VMEM((1, G), jnp.float32),
        ],
    )(h, batch3d, fc_W, fc_b, w1, b1, w2, b2)


def kernel(x, gam0, gam1, gam2, edge_index, batch,
           Wl0, Wr0, bc0, bng0, bnb0,
           Wl1, Wr1, bc1, bng1, bnb1,
           Wl2, Wr2, bc2, bng2, bnb2,
           Wl3, Wr3, bc3, bng3, bnb3,
           fc_W, fc_b, mlp_W1, mlp_b1, mlp_W2, mlp_b2):
    h = jnp.concatenate([x, gam0, gam1, gam2], axis=1)
    src = edge_index[0].astype(jnp.int32)
    dst = edge_index[1].astype(jnp.int32)
    batch3d = batch.astype(jnp.int32).reshape(NB, 1, R)

    bsrc, bdst, cnts, degp = _bucket(src, dst)
    deg = degp.reshape(AGGR)[:N]
    deginv3d = (1.0 / jnp.clip(deg, 1.0)).reshape(NB, 1, R)

    convs = [(Wl0, Wr0, bc0, bng0, bnb0), (Wl1, Wr1, bc1, bng1, bnb1),
             (Wl2, Wr2, bc2, bng2, bnb2), (Wl3, Wr3, bc3, bng3, bnb3)]
    for (Wl, Wr, bc, g, b) in convs:
        agg = _agg(h, bsrc, bdst, cnts).reshape(AGGR, H)
        u, stats = _mm_stats(h, agg, deginv3d, Wl, Wr, bc.reshape(1, H))
        h = _bn_relu(u, stats, g.reshape(1, H), b.reshape(1, H))

    return _pool_head(h, batch3d, fc_W, fc_b.reshape(1, H),
                      mlp_W1, mlp_b1.reshape(1, H),
                      mlp_W2, mlp_b2.reshape(1, OUT))
